# baseline (speedup 1.0000x reference)
import functools

import jax
import jax.numpy as jnp
from jax import lax
from jax.experimental import pallas as pl
from jax.experimental.pallas import tpu as pltpu
from jax.experimental.pallas import tpu_sc as plsc

_B, _NP, _FIN = 4, 100000, 4
_CE, _CB = 80, 128
_H, _W = 128, 128
_HW = _H * _W
_XMIN, _YMIN, _XMAX, _YMAX = -50.0, -50.0, 50.0, 50.0

_NSC = 2
_NT = 16
_CW = 96

_CHUNK1 = 7168
_G1 = 14
_NPP = _G1 * _CHUNK1
_RPT = _NPP // _NT
_SCAT = 128
_ZROWS = 64
_RPB = _HW // _NT
_TRASH = _HW
_ACCR = _HW + 8

_PREC = lax.Precision.HIGHEST


def _stage1_body(pts_ref, ptst_ref, w1_ref, b1_ref, w2_ref, b2_ref,
                 emb_ref, idx_ref):
    pts = pts_ref[0]
    h = jnp.maximum(
        lax.dot_general(pts, w1_ref[...], (((1,), (0,)), ((), ())),
                        preferred_element_type=jnp.float32, precision=_PREC)
        + b1_ref[...], 0.0)
    emb = jnp.maximum(
        lax.dot_general(h, w2_ref[...], (((1,), (0,)), ((), ())),
                        preferred_element_type=jnp.float32, precision=_PREC)
        + b2_ref[...], 0.0)
    ones = jnp.ones((_CHUNK1, 1), jnp.float32)
    pad = jnp.zeros((_CHUNK1, _CW - _CE - 1), jnp.float32)
    emb_ref[0] = jnp.concatenate([emb, ones, pad], axis=1)
    xr = ptst_ref[0, 0:1, :]
    yr = ptst_ref[0, 1:2, :]
    fx = (xr - _XMIN) * (_W / (_XMAX - _XMIN))
    fy = (yr - _YMIN) * (_H / (_YMAX - _YMIN))
    ix = fx.astype(jnp.int32)
    iy = fy.astype(jnp.int32)
    valid = (fx >= 0.0) & (fx < float(_W)) & (fy >= 0.0) & (fy < float(_H))
    idx_ref[0] = jnp.where(valid, iy * _W + ix, _TRASH)


def _stage1(points_p, points_t, W1, b1, W2, b2):
    return pl.pallas_call(
        _stage1_body,
        grid=(_B * _G1,),
        in_specs=[
            pl.BlockSpec((1, _CHUNK1, _FIN), lambda i: (i // _G1, i % _G1, 0)),
            pl.BlockSpec((1, _FIN, _CHUNK1), lambda i: (i // _G1, 0, i % _G1)),
            pl.BlockSpec((_FIN, _CE), lambda i: (0, 0)),
            pl.BlockSpec((1, _CE), lambda i: (0, 0)),
            pl.BlockSpec((_CE, _CE), lambda i: (0, 0)),
            pl.BlockSpec((1, _CE), lambda i: (0, 0)),
        ],
        out_specs=[
            pl.BlockSpec((1, _CHUNK1, _CW), lambda i: (i // _G1, i % _G1, 0)),
            pl.BlockSpec((1, 1, _CHUNK1), lambda i: (i, 0, 0)),
        ],
        out_shape=[
            jax.ShapeDtypeStruct((_B, _NPP, _CW), jnp.float32),
            jax.ShapeDtypeStruct((_B * _G1, 1, _CHUNK1), jnp.int32),
        ],
    )(points_p, points_t, W1, b1, W2, b2)


def _stage2_sc(emb_ext, idx2):
    mesh = plsc.VectorSubcoreMesh(core_axis_name="c", subcore_axis_name="s")

    @functools.partial(
        pl.kernel,
        out_type=jax.ShapeDtypeStruct((_B, _HW, _CW), jnp.float32),
        mesh=mesh,
        scratch_types=[
            pltpu.VMEM((_SCAT,), jnp.int32),
            pltpu.VMEM((_SCAT, _CW), jnp.float32),
            pltpu.VMEM((_ZROWS, _CW), jnp.float32),
            pltpu.VMEM_SHARED((_ACCR, _CW), jnp.float32),
        ],
        compiler_params=pltpu.CompilerParams(use_tc_tiling_on_sc=False),
    )
    def k(emb_hbm, idx_hbm, out_hbm, idx_v, stage_v, zbuf, acc):
        c = lax.axis_index("c")
        s = lax.axis_index("s")

        def _zrow(r, carry):
            for g in range(_CW // 16):
                zbuf[r, pl.ds(g * 16, 16)] = jnp.zeros((16,), jnp.float32)
            return carry

        lax.fori_loop(0, _ZROWS, _zrow, 0)

        for bi in range(_B // _NSC):
            b = c * (_B // _NSC) + bi
            for z in range(_RPB // _ZROWS):
                pltpu.sync_copy(zbuf, acc.at[pl.ds((s * (_RPB // _ZROWS) + z) * _ZROWS, _ZROWS)])
            plsc.subcore_barrier()

            def _chunk(g, carry):
                base = s * _RPT + g * _SCAT
                pltpu.sync_copy(idx_hbm.at[b, pl.ds(base, _SCAT)], idx_v)
                pltpu.sync_copy(emb_hbm.at[b, pl.ds(base, _SCAT)], stage_v)
                pltpu.sync_copy(stage_v, acc.at[idx_v], add=True)
                return carry

            lax.fori_loop(0, _RPT // _SCAT, _chunk, 0)
            plsc.subcore_barrier()
            pltpu.sync_copy(acc.at[pl.ds(s * _RPB, _RPB)],
                            out_hbm.at[b, pl.ds(s * _RPB, _RPB)])

    return k(emb_ext, idx2)


def _stage3_body(acc_ref, wc_ref, bc_ref, gm_ref, bt_ref, waug_ref, s1, m2):
    i = pl.program_id(0)

    @pl.when(i == 0)
    def _init():
        s1[...] = jnp.zeros_like(s1)
        m2[...] = jnp.zeros_like(m2)

    blk = acc_ref[0]
    mean = blk[:, :_CE] * (1.0 / jnp.maximum(blk[:, _CE:_CE + 1], 1.0))
    s1[...] += jnp.sum(mean, axis=0, keepdims=True)
    m2[...] += lax.dot_general(mean, mean, (((0,), (0,)), ((), ())),
                               preferred_element_type=jnp.float32, precision=_PREC)

    @pl.when(i == _B - 1)
    def _fin():
        n = float(_B * _HW)
        gv = s1[...] * (1.0 / n)
        cov = m2[...] * (1.0 / n) - lax.dot_general(
            gv, gv, (((0,), (0,)), ((), ())),
            preferred_element_type=jnp.float32, precision=_PREC)
        wc = wc_ref[...]
        mu_r = lax.dot_general(gv, wc, (((1,), (1,)), ((), ())),
                               preferred_element_type=jnp.float32,
                               precision=_PREC) + bc_ref[...]
        wccov = lax.dot_general(wc, cov, (((1,), (0,)), ((), ())),
                                preferred_element_type=jnp.float32, precision=_PREC)
        e = lax.dot_general(wccov, wc, (((1,), (1,)), ((), ())),
                            preferred_element_type=jnp.float32,
                            precision=_PREC)
        eye = (lax.broadcasted_iota(jnp.int32, (_CB, _CB), 0)
               == lax.broadcasted_iota(jnp.int32, (_CB, _CB), 1)).astype(jnp.float32)
        var_r = jnp.sum(e * eye, axis=0, keepdims=True)
        scale_r = gm_ref[...] * lax.rsqrt(var_r + 1e-5)
        bp_r = (bc_ref[...] - mu_r) * scale_r + bt_ref[...]
        d = eye * scale_r
        wp = lax.dot_general(d, wc, (((1,), (0,)), ((), ())),
                             preferred_element_type=jnp.float32, precision=_PREC)
        bp_c = lax.dot_general(eye, bp_r, (((1,), (1,)), ((), ())),
                               preferred_element_type=jnp.float32,
                               precision=_PREC)
        waug_ref[...] = jnp.concatenate([wp, bp_c], axis=1)


def _stage3(acc, Wc, bc, gamma, beta):
    return pl.pallas_call(
        _stage3_body,
        grid=(_B,),
        in_specs=[
            pl.BlockSpec((1, _HW, _CW), lambda i: (i, 0, 0)),
            pl.BlockSpec((_CB, _CE), lambda i: (0, 0)),
            pl.BlockSpec((1, _CB), lambda i: (0, 0)),
            pl.BlockSpec((1, _CB), lambda i: (0, 0)),
            pl.BlockSpec((1, _CB), lambda i: (0, 0)),
        ],
        out_specs=pl.BlockSpec((_CB, _CE + 1), lambda i: (0, 0)),
        out_shape=jax.ShapeDtypeStruct((_CB, _CE + 1), jnp.float32),
        scratch_shapes=[
            pltpu.VMEM((1, _CE), jnp.float32),
            pltpu.VMEM((_CE, _CE), jnp.float32),
        ],
    )(acc, Wc, bc, gamma, beta)


def _stage4_body(acc_ref, waug_ref, out_ref):
    blk = acc_ref[0]
    mean = blk[:, :_CE] * (1.0 / jnp.maximum(blk[:, _CE:_CE + 1], 1.0))
    aug = jnp.concatenate([mean, jnp.ones((_HW, 1), jnp.float32)], axis=1)
    z = lax.dot_general(waug_ref[...], aug, (((1,), (1,)), ((), ())),
                        preferred_element_type=jnp.float32, precision=_PREC)
    out_ref[0] = jnp.maximum(z, 0.0)


def _stage4(acc, waug):
    return pl.pallas_call(
        _stage4_body,
        grid=(_B,),
        in_specs=[
            pl.BlockSpec((1, _HW, _CW), lambda i: (i, 0, 0)),
            pl.BlockSpec((_CB, _CE + 1), lambda i: (0, 0)),
        ],
        out_specs=pl.BlockSpec((1, _CB, _HW), lambda i: (i, 0, 0)),
        out_shape=jax.ShapeDtypeStruct((_B, _CB, _HW), jnp.float32),
    )(acc, waug)


def kernel(points, W1, b1, W2, b2, Wc, bc, gamma, beta):
    points_p = jnp.pad(points, ((0, 0), (0, _NPP - _NP), (0, 0)),
                       constant_values=1e9)
    points_t = points_p.transpose(0, 2, 1)
    emb_ext, idxflat = _stage1(points_p, points_t, W1, b1.reshape(1, -1),
                               W2, b2.reshape(1, -1))
    acc = _stage2_sc(emb_ext, idxflat.reshape(_B, _NPP))
    waug = _stage3(acc, Wc, bc.reshape(1, -1), gamma.reshape(1, -1),
                   beta.reshape(1, -1))
    out = _stage4(acc, waug)
    return out.reshape(_B, _CB, _H, _W)


# speedup vs baseline: 1.1479x; 1.1479x over previous
import functools

import jax
import jax.numpy as jnp
from jax import lax
from jax.experimental import pallas as pl
from jax.experimental.pallas import tpu as pltpu
from jax.experimental.pallas import tpu_sc as plsc

_B, _NP, _FIN = 4, 100000, 4
_CE, _CB = 80, 128
_H, _W = 128, 128
_HW = _H * _W
_XMIN, _YMIN, _XMAX, _YMAX = -50.0, -50.0, 50.0, 50.0

_NSC = 2
_NT = 16
_CW = 96
_PW = 128

_CHUNK1 = 7168
_G1 = 14
_NPP = _G1 * _CHUNK1
_RPT = _NPP // _NT
_SCAT = 128
_ZROWS = 64
_RPB = _HW // _NT
_TRASH = _HW
_ACCR = _HW + 8

_PREC = lax.Precision.HIGHEST


def _stage1_body(pts_ref, ptst_ref, w1_ref, b1_ref, w2_ref, b2_ref,
                 emb_ref, idx_ref):
    pts = pts_ref[0]
    h = jnp.maximum(
        lax.dot_general(pts, w1_ref[...], (((1,), (0,)), ((), ())),
                        preferred_element_type=jnp.float32, precision=_PREC)
        + b1_ref[...], 0.0)
    emb = jnp.maximum(
        lax.dot_general(h, w2_ref[...], (((1,), (0,)), ((), ())),
                        preferred_element_type=jnp.float32, precision=_PREC)
        + b2_ref[...], 0.0)
    ones = jnp.ones((_CHUNK1, 1), jnp.float32)
    pad = jnp.zeros((_CHUNK1, _PW - _CE - 1), jnp.float32)
    emb_ref[0] = jnp.concatenate([emb, ones, pad], axis=1)
    xr = ptst_ref[0, 0:1, :]
    yr = ptst_ref[0, 1:2, :]
    fx = (xr - _XMIN) * (_W / (_XMAX - _XMIN))
    fy = (yr - _YMIN) * (_H / (_YMAX - _YMIN))
    ix = fx.astype(jnp.int32)
    iy = fy.astype(jnp.int32)
    valid = (fx >= 0.0) & (fx < float(_W)) & (fy >= 0.0) & (fy < float(_H))
    idx_ref[0] = jnp.where(valid, iy * _W + ix, _TRASH)


def _stage1(points_p, points_t, W1, b1, W2, b2):
    return pl.pallas_call(
        _stage1_body,
        grid=(_B * _G1,),
        in_specs=[
            pl.BlockSpec((1, _CHUNK1, _FIN), lambda i: (i // _G1, i % _G1, 0)),
            pl.BlockSpec((1, _FIN, _CHUNK1), lambda i: (i // _G1, 0, i % _G1)),
            pl.BlockSpec((_FIN, _CE), lambda i: (0, 0)),
            pl.BlockSpec((1, _CE), lambda i: (0, 0)),
            pl.BlockSpec((_CE, _CE), lambda i: (0, 0)),
            pl.BlockSpec((1, _CE), lambda i: (0, 0)),
        ],
        out_specs=[
            pl.BlockSpec((1, _CHUNK1, _PW), lambda i: (i // _G1, i % _G1, 0)),
            pl.BlockSpec((1, 1, _CHUNK1), lambda i: (i, 0, 0)),
        ],
        out_shape=[
            jax.ShapeDtypeStruct((_B, _NPP, _PW), jnp.float32),
            jax.ShapeDtypeStruct((_B * _G1, 1, _CHUNK1), jnp.int32),
        ],
    )(points_p, points_t, W1, b1, W2, b2)


def _stage2_sc(emb_ext, idx2):
    mesh = plsc.VectorSubcoreMesh(core_axis_name="c", subcore_axis_name="s")

    @functools.partial(
        pl.kernel,
        out_type=jax.ShapeDtypeStruct((_B, _HW, _CW), jnp.float32),
        mesh=mesh,
        scratch_types=[
            pltpu.VMEM((_SCAT,), jnp.int32),
            pltpu.VMEM((_SCAT, _CW), jnp.float32),
            pltpu.VMEM((_ZROWS, _CW), jnp.float32),
            pltpu.VMEM_SHARED((_ACCR, _CW), jnp.float32),
        ],
        compiler_params=pltpu.CompilerParams(use_tc_tiling_on_sc=False),
    )
    def k(emb_hbm, idx_hbm, out_hbm, idx_v, stage_v, zbuf, acc):
        c = lax.axis_index("c")
        s = lax.axis_index("s")

        def _zrow(r, carry):
            for g in range(_CW // 16):
                zbuf[r, pl.ds(g * 16, 16)] = jnp.zeros((16,), jnp.float32)
            return carry

        lax.fori_loop(0, _ZROWS, _zrow, 0)

        for bi in range(_B // _NSC):
            b = c * (_B // _NSC) + bi
            for z in range(_RPB // _ZROWS):
                pltpu.sync_copy(zbuf, acc.at[pl.ds((s * (_RPB // _ZROWS) + z) * _ZROWS, _ZROWS)])
            plsc.subcore_barrier()

            def _chunk(g, carry):
                base = s * _RPT + g * _SCAT
                pltpu.sync_copy(idx_hbm.at[b, pl.ds(base, _SCAT)], idx_v)
                pltpu.sync_copy(emb_hbm.at[b, pl.ds(base, _SCAT), pl.ds(0, _CW)],
                                stage_v)
                pltpu.sync_copy(stage_v, acc.at[idx_v], add=True)
                return carry

            lax.fori_loop(0, _RPT // _SCAT, _chunk, 0)
            plsc.subcore_barrier()
            pltpu.sync_copy(acc.at[pl.ds(s * _RPB, _RPB)],
                            out_hbm.at[b, pl.ds(s * _RPB, _RPB)])

    return k(emb_ext, idx2)


def _stage3_body(acc_ref, wc_ref, bc_ref, gm_ref, bt_ref, waug_ref, s1, m2):
    i = pl.program_id(0)

    @pl.when(i == 0)
    def _init():
        s1[...] = jnp.zeros_like(s1)
        m2[...] = jnp.zeros_like(m2)

    blk = acc_ref[0]
    mean = blk[:, :_CE] * (1.0 / jnp.maximum(blk[:, _CE:_CE + 1], 1.0))
    s1[...] += jnp.sum(mean, axis=0, keepdims=True)
    m2[...] += lax.dot_general(mean, mean, (((0,), (0,)), ((), ())),
                               preferred_element_type=jnp.float32, precision=_PREC)

    @pl.when(i == _B - 1)
    def _fin():
        n = float(_B * _HW)
        gv = s1[...] * (1.0 / n)
        cov = m2[...] * (1.0 / n) - lax.dot_general(
            gv, gv, (((0,), (0,)), ((), ())),
            preferred_element_type=jnp.float32, precision=_PREC)
        wc = wc_ref[...]
        mu_r = lax.dot_general(gv, wc, (((1,), (1,)), ((), ())),
                               preferred_element_type=jnp.float32,
                               precision=_PREC) + bc_ref[...]
        wccov = lax.dot_general(wc, cov, (((1,), (0,)), ((), ())),
                                preferred_element_type=jnp.float32, precision=_PREC)
        e = lax.dot_general(wccov, wc, (((1,), (1,)), ((), ())),
                            preferred_element_type=jnp.float32,
                            precision=_PREC)
        eye = (lax.broadcasted_iota(jnp.int32, (_CB, _CB), 0)
               == lax.broadcasted_iota(jnp.int32, (_CB, _CB), 1)).astype(jnp.float32)
        var_r = jnp.sum(e * eye, axis=0, keepdims=True)
        scale_r = gm_ref[...] * lax.rsqrt(var_r + 1e-5)
        bp_r = (bc_ref[...] - mu_r) * scale_r + bt_ref[...]
        d = eye * scale_r
        wp = lax.dot_general(d, wc, (((1,), (0,)), ((), ())),
                             preferred_element_type=jnp.float32, precision=_PREC)
        bp_c = lax.dot_general(eye, bp_r, (((1,), (1,)), ((), ())),
                               preferred_element_type=jnp.float32,
                               precision=_PREC)
        waug_ref[...] = jnp.concatenate([wp, bp_c], axis=1)


def _stage3(acc, Wc, bc, gamma, beta):
    return pl.pallas_call(
        _stage3_body,
        grid=(_B,),
        in_specs=[
            pl.BlockSpec((1, _HW, _CW), lambda i: (i, 0, 0)),
            pl.BlockSpec((_CB, _CE), lambda i: (0, 0)),
            pl.BlockSpec((1, _CB), lambda i: (0, 0)),
            pl.BlockSpec((1, _CB), lambda i: (0, 0)),
            pl.BlockSpec((1, _CB), lambda i: (0, 0)),
        ],
        out_specs=pl.BlockSpec((_CB, _CE + 1), lambda i: (0, 0)),
        out_shape=jax.ShapeDtypeStruct((_CB, _CE + 1), jnp.float32),
        scratch_shapes=[
            pltpu.VMEM((1, _CE), jnp.float32),
            pltpu.VMEM((_CE, _CE), jnp.float32),
        ],
    )(acc, Wc, bc, gamma, beta)


def _stage4_body(acc_ref, waug_ref, out_ref):
    blk = acc_ref[0]
    mean = blk[:, :_CE] * (1.0 / jnp.maximum(blk[:, _CE:_CE + 1], 1.0))
    aug = jnp.concatenate([mean, jnp.ones((_HW, 1), jnp.float32)], axis=1)
    z = lax.dot_general(waug_ref[...], aug, (((1,), (1,)), ((), ())),
                        preferred_element_type=jnp.float32, precision=_PREC)
    out_ref[0] = jnp.maximum(z, 0.0)


def _stage4(acc, waug):
    return pl.pallas_call(
        _stage4_body,
        grid=(_B,),
        in_specs=[
            pl.BlockSpec((1, _HW, _CW), lambda i: (i, 0, 0)),
            pl.BlockSpec((_CB, _CE + 1), lambda i: (0, 0)),
        ],
        out_specs=pl.BlockSpec((1, _CB, _HW), lambda i: (i, 0, 0)),
        out_shape=jax.ShapeDtypeStruct((_B, _CB, _HW), jnp.float32),
    )(acc, waug)


def kernel(points, W1, b1, W2, b2, Wc, bc, gamma, beta):
    points_p = jnp.pad(points, ((0, 0), (0, _NPP - _NP), (0, 0)),
                       constant_values=1e9)
    points_t = points_p.transpose(0, 2, 1)
    emb_ext, idxflat = _stage1(points_p, points_t, W1, b1.reshape(1, -1),
                               W2, b2.reshape(1, -1))
    acc = _stage2_sc(emb_ext, idxflat.reshape(_B, _NPP))
    waug = _stage3(acc, Wc, bc.reshape(1, -1), gamma.reshape(1, -1),
                   beta.reshape(1, -1))
    out = _stage4(acc, waug)
    return out.reshape(_B, _CB, _H, _W)


# speedup vs baseline: 1.7333x; 1.5099x over previous
import functools

import jax
import jax.numpy as jnp
from jax import lax
from jax.experimental import pallas as pl
from jax.experimental.pallas import tpu as pltpu
from jax.experimental.pallas import tpu_sc as plsc

_B, _NP, _FIN = 4, 100000, 4
_CE, _CB = 80, 128
_H, _W = 128, 128
_HW = _H * _W
_XMIN, _YMIN, _XMAX, _YMAX = -50.0, -50.0, 50.0, 50.0

_NSC = 2
_NT = 16
_CW = 96
_PW = 128

_CHUNK1 = 7168
_G1 = 14
_NPP = _G1 * _CHUNK1
_RPT = _NPP // _NT
_SCAT = 128
_ZROWS = 64
_RPB = _HW // _NT
_TRASH = _HW
_ACCR = _HW + 8

_PREC = lax.Precision.HIGHEST


def _stage1_body(pts_ref, ptst_ref, w1_ref, b1_ref, w2_ref, b2_ref,
                 emb_ref, idx_ref):
    pts = pts_ref[0]
    h = jnp.maximum(
        lax.dot_general(pts, w1_ref[...], (((1,), (0,)), ((), ())),
                        preferred_element_type=jnp.float32, precision=_PREC)
        + b1_ref[...], 0.0)
    emb = jnp.maximum(
        lax.dot_general(h, w2_ref[...], (((1,), (0,)), ((), ())),
                        preferred_element_type=jnp.float32, precision=_PREC)
        + b2_ref[...], 0.0)
    ones = jnp.ones((_CHUNK1, 1), jnp.float32)
    pad = jnp.zeros((_CHUNK1, _PW - _CE - 1), jnp.float32)
    emb_ref[0] = jnp.concatenate([emb, ones, pad], axis=1)
    xr = ptst_ref[0, 0:1, :]
    yr = ptst_ref[0, 1:2, :]
    fx = (xr - _XMIN) * (_W / (_XMAX - _XMIN))
    fy = (yr - _YMIN) * (_H / (_YMAX - _YMIN))
    ix = fx.astype(jnp.int32)
    iy = fy.astype(jnp.int32)
    valid = (fx >= 0.0) & (fx < float(_W)) & (fy >= 0.0) & (fy < float(_H))
    idx_ref[0] = jnp.where(valid, iy * _W + ix, _TRASH)


def _stage1(points_p, points_t, W1, b1, W2, b2, nb):
    return pl.pallas_call(
        _stage1_body,
        grid=(nb * _G1,),
        in_specs=[
            pl.BlockSpec((1, _CHUNK1, _FIN), lambda i: (i // _G1, i % _G1, 0)),
            pl.BlockSpec((1, _FIN, _CHUNK1), lambda i: (i // _G1, 0, i % _G1)),
            pl.BlockSpec((_FIN, _CE), lambda i: (0, 0)),
            pl.BlockSpec((1, _CE), lambda i: (0, 0)),
            pl.BlockSpec((_CE, _CE), lambda i: (0, 0)),
            pl.BlockSpec((1, _CE), lambda i: (0, 0)),
        ],
        out_specs=[
            pl.BlockSpec((1, _CHUNK1, _PW), lambda i: (i // _G1, i % _G1, 0)),
            pl.BlockSpec((1, 1, _CHUNK1), lambda i: (i, 0, 0)),
        ],
        out_shape=[
            jax.ShapeDtypeStruct((nb, _NPP, _PW), jnp.float32),
            jax.ShapeDtypeStruct((nb * _G1, 1, _CHUNK1), jnp.int32),
        ],
    )(points_p, points_t, W1, b1, W2, b2)


def _stage2_sc(emb_ext, idx2, nb):
    mesh = plsc.VectorSubcoreMesh(core_axis_name="c", subcore_axis_name="s")

    @functools.partial(
        pl.kernel,
        out_type=jax.ShapeDtypeStruct((nb, _HW, _CW), jnp.float32),
        mesh=mesh,
        scratch_types=[
            pltpu.VMEM((_SCAT,), jnp.int32),
            pltpu.VMEM((_SCAT, _CW), jnp.float32),
            pltpu.VMEM((_ZROWS, _CW), jnp.float32),
            pltpu.VMEM_SHARED((_ACCR, _CW), jnp.float32),
        ],
        compiler_params=pltpu.CompilerParams(use_tc_tiling_on_sc=False),
    )
    def k(emb_hbm, idx_hbm, out_hbm, idx_v, stage_v, zbuf, acc):
        c = lax.axis_index("c")
        s = lax.axis_index("s")

        def _zrow(r, carry):
            for g in range(_CW // 16):
                zbuf[r, pl.ds(g * 16, 16)] = jnp.zeros((16,), jnp.float32)
            return carry

        lax.fori_loop(0, _ZROWS, _zrow, 0)

        for bi in range(nb // _NSC):
            b = c * (nb // _NSC) + bi
            for z in range(_RPB // _ZROWS):
                pltpu.sync_copy(zbuf, acc.at[pl.ds((s * (_RPB // _ZROWS) + z) * _ZROWS, _ZROWS)])
            plsc.subcore_barrier()

            def _chunk(g, carry):
                base = s * _RPT + g * _SCAT
                pltpu.sync_copy(idx_hbm.at[b, pl.ds(base, _SCAT)], idx_v)
                pltpu.sync_copy(emb_hbm.at[b, pl.ds(base, _SCAT), pl.ds(0, _CW)],
                                stage_v)
                pltpu.sync_copy(stage_v, acc.at[idx_v], add=True)
                return carry

            lax.fori_loop(0, _RPT // _SCAT, _chunk, 0)
            plsc.subcore_barrier()
            pltpu.sync_copy(acc.at[pl.ds(s * _RPB, _RPB)],
                            out_hbm.at[b, pl.ds(s * _RPB, _RPB)])

    return k(emb_ext, idx2)


def _stage3_body(acc_ref, wc_ref, bc_ref, gm_ref, bt_ref, waug_ref, s1, m2):
    i = pl.program_id(0)

    @pl.when(i == 0)
    def _init():
        s1[...] = jnp.zeros_like(s1)
        m2[...] = jnp.zeros_like(m2)

    blk = acc_ref[0]
    mean = blk[:, :_CE] * (1.0 / jnp.maximum(blk[:, _CE:_CE + 1], 1.0))
    s1[...] += jnp.sum(mean, axis=0, keepdims=True)
    m2[...] += lax.dot_general(mean, mean, (((0,), (0,)), ((), ())),
                               preferred_element_type=jnp.float32, precision=_PREC)

    @pl.when(i == _B - 1)
    def _fin():
        n = float(_B * _HW)
        gv = s1[...] * (1.0 / n)
        cov = m2[...] * (1.0 / n) - lax.dot_general(
            gv, gv, (((0,), (0,)), ((), ())),
            preferred_element_type=jnp.float32, precision=_PREC)
        wc = wc_ref[...]
        mu_r = lax.dot_general(gv, wc, (((1,), (1,)), ((), ())),
                               preferred_element_type=jnp.float32,
                               precision=_PREC) + bc_ref[...]
        wccov = lax.dot_general(wc, cov, (((1,), (0,)), ((), ())),
                                preferred_element_type=jnp.float32, precision=_PREC)
        e = lax.dot_general(wccov, wc, (((1,), (1,)), ((), ())),
                            preferred_element_type=jnp.float32,
                            precision=_PREC)
        eye = (lax.broadcasted_iota(jnp.int32, (_CB, _CB), 0)
               == lax.broadcasted_iota(jnp.int32, (_CB, _CB), 1)).astype(jnp.float32)
        var_r = jnp.sum(e * eye, axis=0, keepdims=True)
        scale_r = gm_ref[...] * lax.rsqrt(var_r + 1e-5)
        bp_r = (bc_ref[...] - mu_r) * scale_r + bt_ref[...]
        d = eye * scale_r
        wp = lax.dot_general(d, wc, (((1,), (0,)), ((), ())),
                             preferred_element_type=jnp.float32, precision=_PREC)
        bp_c = lax.dot_general(eye, bp_r, (((1,), (1,)), ((), ())),
                               preferred_element_type=jnp.float32,
                               precision=_PREC)
        waug_ref[...] = jnp.concatenate([wp, bp_c], axis=1)


def _stage3(acc, Wc, bc, gamma, beta):
    return pl.pallas_call(
        _stage3_body,
        grid=(_B,),
        in_specs=[
            pl.BlockSpec((1, _HW, _CW), lambda i: (i, 0, 0)),
            pl.BlockSpec((_CB, _CE), lambda i: (0, 0)),
            pl.BlockSpec((1, _CB), lambda i: (0, 0)),
            pl.BlockSpec((1, _CB), lambda i: (0, 0)),
            pl.BlockSpec((1, _CB), lambda i: (0, 0)),
        ],
        out_specs=pl.BlockSpec((_CB, _CE + 1), lambda i: (0, 0)),
        out_shape=jax.ShapeDtypeStruct((_CB, _CE + 1), jnp.float32),
        scratch_shapes=[
            pltpu.VMEM((1, _CE), jnp.float32),
            pltpu.VMEM((_CE, _CE), jnp.float32),
        ],
    )(acc, Wc, bc, gamma, beta)


def _stage4_body(acc_ref, waug_ref, out_ref):
    blk = acc_ref[0]
    mean = blk[:, :_CE] * (1.0 / jnp.maximum(blk[:, _CE:_CE + 1], 1.0))
    aug = jnp.concatenate([mean, jnp.ones((_HW, 1), jnp.float32)], axis=1)
    z = lax.dot_general(waug_ref[...], aug, (((1,), (1,)), ((), ())),
                        preferred_element_type=jnp.float32, precision=_PREC)
    out_ref[0] = jnp.maximum(z, 0.0)


def _stage4(acc, waug):
    return pl.pallas_call(
        _stage4_body,
        grid=(_B,),
        in_specs=[
            pl.BlockSpec((1, _HW, _CW), lambda i: (i, 0, 0)),
            pl.BlockSpec((_CB, _CE + 1), lambda i: (0, 0)),
        ],
        out_specs=pl.BlockSpec((1, _CB, _HW), lambda i: (i, 0, 0)),
        out_shape=jax.ShapeDtypeStruct((_B, _CB, _HW), jnp.float32),
    )(acc, waug)


def kernel(points, W1, b1, W2, b2, Wc, bc, gamma, beta):
    points_p = jnp.pad(points, ((0, 0), (0, _NPP - _NP), (0, 0)),
                       constant_values=1e9)
    points_t = points_p.transpose(0, 2, 1)
    b1r = b1.reshape(1, -1)
    b2r = b2.reshape(1, -1)
    nb = _NSC
    accs = []
    for k in range(_B // nb):
        sl = slice(k * nb, (k + 1) * nb)
        emb_ext, idxflat = _stage1(points_p[sl], points_t[sl], W1, b1r, W2, b2r,
                                   nb)
        accs.append(_stage2_sc(emb_ext, idxflat.reshape(nb, _NPP), nb))
    acc = jnp.concatenate(accs, axis=0)
    waug = _stage3(acc, Wc, bc.reshape(1, -1), gamma.reshape(1, -1),
                   beta.reshape(1, -1))
    out = _stage4(acc, waug)
    return out.reshape(_B, _CB, _H, _W)


# speedup vs baseline: 3.0589x; 1.7648x over previous
import functools

import jax
import jax.numpy as jnp
from jax import lax
from jax.experimental import pallas as pl
from jax.experimental.pallas import tpu as pltpu
from jax.experimental.pallas import tpu_sc as plsc

_B, _NP, _FIN = 4, 100000, 4
_CE, _CB = 80, 128
_H, _W = 128, 128
_HW = _H * _W
_XMIN, _YMIN, _XMAX, _YMAX = -50.0, -50.0, 50.0, 50.0

_NSC = 2
_NT = 16
_CW = 96
_PW = 128

_CHUNK1 = 7168
_G1 = 14
_NPP = _G1 * _CHUNK1
_RPT = _NPP // _NT
_SCAT = 128
_ZROWS = 64
_RPB = _HW // _NT
_TRASH = _HW
_ACCR = _HW + 8

_PREC = lax.Precision.HIGHEST


def _stage1_body(pts_ref, ptst_ref, w1_ref, b1_ref, w2_ref, b2_ref,
                 emb_ref, idx_ref):
    pts = pts_ref[0]
    h = jnp.maximum(
        lax.dot_general(pts, w1_ref[...], (((1,), (0,)), ((), ())),
                        preferred_element_type=jnp.float32)
        + b1_ref[...], 0.0)
    emb = jnp.maximum(
        lax.dot_general(h, w2_ref[...], (((1,), (0,)), ((), ())),
                        preferred_element_type=jnp.float32)
        + b2_ref[...], 0.0)
    ones = jnp.ones((_CHUNK1, 1), jnp.float32)
    pad = jnp.zeros((_CHUNK1, _PW - _CE - 1), jnp.float32)
    emb_ref[0] = jnp.concatenate([emb, ones, pad], axis=1)
    xr = ptst_ref[0, 0:1, :]
    yr = ptst_ref[0, 1:2, :]
    fx = (xr - _XMIN) * (_W / (_XMAX - _XMIN))
    fy = (yr - _YMIN) * (_H / (_YMAX - _YMIN))
    ix = fx.astype(jnp.int32)
    iy = fy.astype(jnp.int32)
    valid = (fx >= 0.0) & (fx < float(_W)) & (fy >= 0.0) & (fy < float(_H))
    idx_ref[0] = jnp.where(valid, iy * _W + ix, _TRASH)


def _stage1(points_p, points_t, W1, b1, W2, b2, nb):
    return pl.pallas_call(
        _stage1_body,
        grid=(nb * _G1,),
        in_specs=[
            pl.BlockSpec((1, _CHUNK1, _FIN), lambda i: (i // _G1, i % _G1, 0)),
            pl.BlockSpec((1, _FIN, _CHUNK1), lambda i: (i // _G1, 0, i % _G1)),
            pl.BlockSpec((_FIN, _CE), lambda i: (0, 0)),
            pl.BlockSpec((1, _CE), lambda i: (0, 0)),
            pl.BlockSpec((_CE, _CE), lambda i: (0, 0)),
            pl.BlockSpec((1, _CE), lambda i: (0, 0)),
        ],
        out_specs=[
            pl.BlockSpec((1, _CHUNK1, _PW), lambda i: (i // _G1, i % _G1, 0)),
            pl.BlockSpec((1, 1, _CHUNK1), lambda i: (i, 0, 0)),
        ],
        out_shape=[
            jax.ShapeDtypeStruct((nb, _NPP, _PW), jnp.float32),
            jax.ShapeDtypeStruct((nb * _G1, 1, _CHUNK1), jnp.int32),
        ],
    )(points_p, points_t, W1, b1, W2, b2)


def _stage2_sc(emb_ext, idx2, nb):
    mesh = plsc.VectorSubcoreMesh(core_axis_name="c", subcore_axis_name="s")

    @functools.partial(
        pl.kernel,
        out_type=jax.ShapeDtypeStruct((nb, _HW, _CW), jnp.float32),
        mesh=mesh,
        scratch_types=[
            pltpu.VMEM((_SCAT,), jnp.int32),
            pltpu.VMEM((_SCAT, _CW), jnp.float32),
            pltpu.VMEM((_ZROWS, _CW), jnp.float32),
            pltpu.VMEM_SHARED((_ACCR, _CW), jnp.float32),
        ],
        compiler_params=pltpu.CompilerParams(use_tc_tiling_on_sc=False),
    )
    def k(emb_hbm, idx_hbm, out_hbm, idx_v, stage_v, zbuf, acc):
        c = lax.axis_index("c")
        s = lax.axis_index("s")

        def _zrow(r, carry):
            for g in range(_CW // 16):
                zbuf[r, pl.ds(g * 16, 16)] = jnp.zeros((16,), jnp.float32)
            return carry

        lax.fori_loop(0, _ZROWS, _zrow, 0)

        for bi in range(nb // _NSC):
            b = c * (nb // _NSC) + bi
            for z in range(_RPB // _ZROWS):
                pltpu.sync_copy(zbuf, acc.at[pl.ds((s * (_RPB // _ZROWS) + z) * _ZROWS, _ZROWS)])
            plsc.subcore_barrier()

            def _chunk(g, carry):
                base = s * _RPT + g * _SCAT
                pltpu.sync_copy(idx_hbm.at[b, pl.ds(base, _SCAT)], idx_v)
                pltpu.sync_copy(emb_hbm.at[b, pl.ds(base, _SCAT), pl.ds(0, _CW)],
                                stage_v)
                pltpu.sync_copy(stage_v, acc.at[idx_v], add=True)
                return carry

            lax.fori_loop(0, _RPT // _SCAT, _chunk, 0)
            plsc.subcore_barrier()
            pltpu.sync_copy(acc.at[pl.ds(s * _RPB, _RPB)],
                            out_hbm.at[b, pl.ds(s * _RPB, _RPB)])

    return k(emb_ext, idx2)


def _stage3_body(acc_ref, wc_ref, bc_ref, gm_ref, bt_ref, waug_ref, s1, m2):
    i = pl.program_id(0)

    @pl.when(i == 0)
    def _init():
        s1[...] = jnp.zeros_like(s1)
        m2[...] = jnp.zeros_like(m2)

    blk = acc_ref[0]
    mean = blk[:, :_CE] * (1.0 / jnp.maximum(blk[:, _CE:_CE + 1], 1.0))
    s1[...] += jnp.sum(mean, axis=0, keepdims=True)
    m2[...] += lax.dot_general(mean, mean, (((0,), (0,)), ((), ())),
                               preferred_element_type=jnp.float32, precision=_PREC)

    @pl.when(i == _B - 1)
    def _fin():
        n = float(_B * _HW)
        gv = s1[...] * (1.0 / n)
        cov = m2[...] * (1.0 / n) - lax.dot_general(
            gv, gv, (((0,), (0,)), ((), ())),
            preferred_element_type=jnp.float32, precision=_PREC)
        wc = wc_ref[...]
        mu_r = lax.dot_general(gv, wc, (((1,), (1,)), ((), ())),
                               preferred_element_type=jnp.float32,
                               precision=_PREC) + bc_ref[...]
        wccov = lax.dot_general(wc, cov, (((1,), (0,)), ((), ())),
                                preferred_element_type=jnp.float32, precision=_PREC)
        e = lax.dot_general(wccov, wc, (((1,), (1,)), ((), ())),
                            preferred_element_type=jnp.float32,
                            precision=_PREC)
        eye = (lax.broadcasted_iota(jnp.int32, (_CB, _CB), 0)
               == lax.broadcasted_iota(jnp.int32, (_CB, _CB), 1)).astype(jnp.float32)
        var_r = jnp.sum(e * eye, axis=0, keepdims=True)
        scale_r = gm_ref[...] * lax.rsqrt(var_r + 1e-5)
        bp_r = (bc_ref[...] - mu_r) * scale_r + bt_ref[...]
        d = eye * scale_r
        wp = lax.dot_general(d, wc, (((1,), (0,)), ((), ())),
                             preferred_element_type=jnp.float32, precision=_PREC)
        bp_c = lax.dot_general(eye, bp_r, (((1,), (1,)), ((), ())),
                               preferred_element_type=jnp.float32,
                               precision=_PREC)
        waug_ref[...] = jnp.concatenate([wp, bp_c], axis=1)


def _stage3(acc, Wc, bc, gamma, beta):
    return pl.pallas_call(
        _stage3_body,
        grid=(_B,),
        in_specs=[
            pl.BlockSpec((1, _HW, _CW), lambda i: (i, 0, 0)),
            pl.BlockSpec((_CB, _CE), lambda i: (0, 0)),
            pl.BlockSpec((1, _CB), lambda i: (0, 0)),
            pl.BlockSpec((1, _CB), lambda i: (0, 0)),
            pl.BlockSpec((1, _CB), lambda i: (0, 0)),
        ],
        out_specs=pl.BlockSpec((_CB, _CE + 1), lambda i: (0, 0)),
        out_shape=jax.ShapeDtypeStruct((_CB, _CE + 1), jnp.float32),
        scratch_shapes=[
            pltpu.VMEM((1, _CE), jnp.float32),
            pltpu.VMEM((_CE, _CE), jnp.float32),
        ],
    )(acc, Wc, bc, gamma, beta)


def _stage4_body(acc_ref, waug_ref, out_ref):
    blk = acc_ref[0]
    mean = blk[:, :_CE] * (1.0 / jnp.maximum(blk[:, _CE:_CE + 1], 1.0))
    aug = jnp.concatenate([mean, jnp.ones((_HW, 1), jnp.float32)], axis=1)
    z = lax.dot_general(waug_ref[...], aug, (((1,), (1,)), ((), ())),
                        preferred_element_type=jnp.float32, precision=_PREC)
    out_ref[0] = jnp.maximum(z, 0.0)


def _stage4(acc, waug):
    return pl.pallas_call(
        _stage4_body,
        grid=(_B,),
        in_specs=[
            pl.BlockSpec((1, _HW, _CW), lambda i: (i, 0, 0)),
            pl.BlockSpec((_CB, _CE + 1), lambda i: (0, 0)),
        ],
        out_specs=pl.BlockSpec((1, _CB, _HW), lambda i: (i, 0, 0)),
        out_shape=jax.ShapeDtypeStruct((_B, _CB, _HW), jnp.float32),
    )(acc, waug)


def kernel(points, W1, b1, W2, b2, Wc, bc, gamma, beta):
    points_p = jnp.pad(points, ((0, 0), (0, _NPP - _NP), (0, 0)),
                       constant_values=1e9)
    points_t = points_p.transpose(0, 2, 1)
    b1r = b1.reshape(1, -1)
    b2r = b2.reshape(1, -1)
    nb = _NSC
    accs = []
    for k in range(_B // nb):
        sl = slice(k * nb, (k + 1) * nb)
        emb_ext, idxflat = _stage1(points_p[sl], points_t[sl], W1, b1r, W2, b2r,
                                   nb)
        accs.append(_stage2_sc(emb_ext, idxflat.reshape(nb, _NPP), nb))
    acc = jnp.concatenate(accs, axis=0)
    waug = _stage3(acc, Wc, bc.reshape(1, -1), gamma.reshape(1, -1),
                   beta.reshape(1, -1))
    out = _stage4(acc, waug)
    return out.reshape(_B, _CB, _H, _W)
